# baseline (speedup 1.0000x reference)
import functools

import jax
import jax.numpy as jnp
from jax import lax
from jax.experimental import pallas as pl
from jax.experimental.pallas import tpu as pltpu

_LANES = 128


def _stem_kernel(xs_ref, w_ref, b_ref, o_ref, patch_ref, *, Ho, W2, Hrows):
    for ky in range(7):
        base = (ky % 2) * Hrows + ky // 2
        patch_ref[:, :, 36 * ky:36 * ky + 36] = xs_ref[0, 0, base:base + Ho]

    pat = patch_ref[...].reshape(Ho * W2, 252)

    y = jnp.dot(pat, w_ref[0], preferred_element_type=jnp.float32) + b_ref[0]
    y = jnp.maximum(y, 0.0)
    y3 = y.reshape(Ho, W2, _LANES)

    y_sw = pltpu.roll(y3, shift=64, axis=2)
    prev_odd = pltpu.roll(y_sw, shift=1, axis=1)
    q = lax.broadcasted_iota(jnp.int32, (Ho, W2, 1), 1)
    prev_odd = jnp.where(q == 0, 0.0, prev_odd)
    b = jnp.maximum(jnp.maximum(y3, y_sw), prev_odd)

    b4 = b.reshape(Ho // 2, 2, W2, _LANES)
    be = b4[:, 0]
    bo = b4[:, 1]
    bo_up = pltpu.roll(bo, shift=1, axis=0)
    i = lax.broadcasted_iota(jnp.int32, (Ho // 2, 1, 1), 0)
    bo_up = jnp.where(i == 0, 0.0, bo_up)
    pooled = jnp.maximum(jnp.maximum(be, bo), bo_up)

    o_ref[0, 0] = pooled.astype(o_ref.dtype)


def _fold_weights(conv_w, gamma, beta, mean, var, eps=1e-5):
    scale = gamma / jnp.sqrt(var + eps)
    bias = beta - mean * scale
    w = conv_w.astype(jnp.float32) * scale[:, None, None, None]
    O, C = w.shape[0], w.shape[1]
    wt = jnp.transpose(w, (2, 3, 1, 0))
    wk = jnp.zeros((7, 12, C, _LANES), jnp.float32)
    wk = wk.at[:, 0:7, :, 0:O].set(wt)
    wk = wk.at[:, 2:9, :, 64:64 + O].set(wt)
    wk = wk.reshape(7 * 12 * C, _LANES).astype(jnp.bfloat16)
    b128 = (jnp.zeros((_LANES,), jnp.float32)
            .at[0:O].set(bias).at[64:64 + O].set(bias))
    return wk, b128.reshape(1, _LANES)


def _stem_pair(x_clic, x_derm, params_cli, params_derm):
    N, C, H, W = x_clic.shape
    O = params_cli["conv1_w"].shape[0]
    Ho, Wo = H // 2, W // 2
    Hp, W2 = Ho // 2, Wo // 2
    Hrows = (H + 6) // 2

    x = jnp.stack([x_clic, x_derm], axis=0)
    x = jnp.transpose(x, (0, 1, 3, 4, 2))
    xp = jnp.pad(x, ((0, 0), (0, 0), (3, 3), (3, 5), (0, 0)))
    xp = xp.astype(jnp.bfloat16)
    xp = xp.reshape(2, N, Hrows, 2, (W + 8) * C)
    xp = jnp.transpose(xp, (0, 1, 3, 2, 4))
    v = xp.reshape(2, N, 2, Hrows, (W + 8) // 4, 4 * C)
    xs = jnp.concatenate(
        [v[..., 0:W2, :], v[..., 1:W2 + 1, :], v[..., 2:W2 + 2, :]], axis=-1)
    xs = xs.reshape(2, N, 2 * Hrows, W2, 12 * C)

    w_c, b_c = _fold_weights(params_cli["conv1_w"], params_cli["bn_gamma"],
                             params_cli["bn_beta"], params_cli["bn_mean"],
                             params_cli["bn_var"])
    w_d, b_d = _fold_weights(params_derm["conv1_w"], params_derm["bn_gamma"],
                             params_derm["bn_beta"], params_derm["bn_mean"],
                             params_derm["bn_var"])
    w = jnp.stack([w_c, w_d], axis=0)
    bias = jnp.stack([b_c, b_d], axis=0)

    body = functools.partial(_stem_kernel, Ho=Ho, W2=W2, Hrows=Hrows)
    out = pl.pallas_call(
        body,
        out_shape=jax.ShapeDtypeStruct((2, N, Hp, W2, _LANES), jnp.bfloat16),
        grid=(2, N),
        in_specs=[
            pl.BlockSpec((1, 1, 2 * Hrows, W2, 36), lambda br, n: (br, n, 0, 0, 0)),
            pl.BlockSpec((1, 252, _LANES), lambda br, n: (br, 0, 0)),
            pl.BlockSpec((1, 1, _LANES), lambda br, n: (br, 0, 0)),
        ],
        out_specs=pl.BlockSpec((1, 1, Hp, W2, _LANES),
                               lambda br, n: (br, n, 0, 0, 0)),
        scratch_shapes=[pltpu.VMEM((Ho, W2, 252), jnp.bfloat16)],
        compiler_params=pltpu.CompilerParams(
            dimension_semantics=("parallel", "parallel"),
            vmem_limit_bytes=64 * 1024 * 1024,
        ),
    )(xs, w, bias)

    out = out[..., :O]
    out = jnp.transpose(out, (0, 1, 4, 2, 3))
    return out[0], out[1]


def kernel(x_clic, x_derm,
           conv1_w_cli, bn_gamma_cli, bn_beta_cli, bn_mean_cli, bn_var_cli,
           conv1_w_derm, bn_gamma_derm, bn_beta_derm, bn_mean_derm, bn_var_derm):
    params_cli = {
        "conv1_w": conv1_w_cli,
        "bn_gamma": bn_gamma_cli,
        "bn_beta": bn_beta_cli,
        "bn_mean": bn_mean_cli,
        "bn_var": bn_var_cli,
    }
    params_derm = {
        "conv1_w": conv1_w_derm,
        "bn_gamma": bn_gamma_derm,
        "bn_beta": bn_beta_derm,
        "bn_mean": bn_mean_derm,
        "bn_var": bn_var_derm,
    }
    return _stem_pair(x_clic, x_derm, params_cli, params_derm)
